# Initial kernel scaffold; baseline (speedup 1.0000x reference)
import jax
import jax.numpy as jnp
from jax.experimental import pallas as pl


def kernel(epoch, hidden_states, _X, edge_index, edge_type, _node_type, _node_feature_extra, params):
    raise NotImplementedError("write your pallas kernel here")


# speedup vs baseline: 1.2964x; 1.2964x over previous
import functools

import jax
import jax.numpy as jnp
from jax import lax
from jax.experimental import pallas as pl
from jax.experimental.pallas import tpu as pltpu

HID = 128
N_NTYPE = 4
N_ETYPE = 38
HEADS = 4
DPH = HID // HEADS
H_S = 1024
EP_1 = 1.0
BS = 10
N_NODE = 1000
SEQ_LEN = 128
N_EDGES = 320000


def _apply(p, x):
    return x @ p["w"] + p["b"]


def _post_kernel(aggr_ref, w1_ref, b1_ref, w2_ref, b2_ref, wq_ref, bq_ref,
                 km_ref, vm_ref, noise_ref, x_ref):
    aggr = aggr_ref[...]
    h = jnp.maximum(jnp.dot(aggr, w1_ref[...], preferred_element_type=jnp.float32) + b1_ref[...], 0.0)
    out = jnp.dot(h, w2_ref[...], preferred_element_type=jnp.float32) + b2_ref[...]
    rn = jnp.sum(out * out, axis=1, keepdims=True)
    sens = jnp.sqrt(jnp.max(rn))
    g = out * 0.5 * (1.0 + jnp.tanh(0.7978845608028654 * (out + 0.044715 * out * out * out)))
    q = jnp.dot(g, wq_ref[...], preferred_element_type=jnp.float32) + bq_ref[...]
    qn = q / jnp.clip(jnp.sqrt(jnp.sum(q * q, axis=1, keepdims=True)), 1e-12, None)
    scale = sens * (4.0 / EP_1)
    for b in range(BS):
        sl = slice(b * N_NODE, (b + 1) * N_NODE)
        km_b = km_ref[b, :]
        vm_b = vm_ref[b, :]
        att = jnp.sum(qn[sl] * km_b[None, :], axis=1, keepdims=True) * (1.0 / jnp.sqrt(float(HID)))
        att = att + noise_ref[sl] * scale
        x_ref[sl, :] = att * vm_b[None, :]


def _post_attention(aggr, lp, wq, km, vm, noise):
    n = aggr.shape[0]
    return pl.pallas_call(
        _post_kernel,
        out_shape=jax.ShapeDtypeStruct((n, HID), jnp.float32),
    )(aggr, lp["mlp1"]["w"], lp["mlp1"]["b"], lp["mlp2"]["w"], lp["mlp2"]["b"],
      wq["w"], wq["b"], km, vm, noise)


def kernel(epoch, hidden_states, _X, edge_index, edge_type, _node_type, _node_feature_extra, params):
    N = _X.shape[0]

    loop = jnp.arange(N, dtype=edge_index.dtype)
    src = jnp.concatenate([edge_index[0], loop])
    dst = jnp.concatenate([edge_index[1], loop])

    edge_vec = jax.nn.one_hot(edge_type, N_ETYPE + 1, dtype=jnp.float32)
    self_edge_vec = jnp.zeros((N, N_ETYPE + 1), jnp.float32).at[:, N_ETYPE].set(1.0)
    head_oh = jax.nn.one_hot(_node_type[edge_index[0]], N_NTYPE, dtype=jnp.float32)
    tail_oh = jax.nn.one_hot(_node_type[edge_index[1]], N_NTYPE, dtype=jnp.float32)
    ht = jnp.concatenate([head_oh, tail_oh], axis=1)
    self_oh = jax.nn.one_hot(_node_type, N_NTYPE, dtype=jnp.float32)
    self_ht = jnp.concatenate([self_oh, self_oh], axis=1)
    ev = jnp.concatenate([edge_vec, self_edge_vec], axis=0)
    htv = jnp.concatenate([ht, self_ht], axis=0)
    ee = params["edge_enc"]
    edge_emb = _apply(ee["l2"], jax.nn.relu(_apply(ee["l1"], jnp.concatenate([ev, htv], axis=1))))

    context = _apply(params["proj_out"], hidden_states[:, 0, :])
    km = _apply(params["Wk"], context)
    vm = _apply(params["Wv"], context)

    noises = [
        jax.random.laplace(jax.random.fold_in(jax.random.key(1), li), (BS, N_NODE, 1), jnp.float32)
        .reshape(N, 1)
        for li in range(5)
    ]

    nfe = _node_feature_extra
    X = _X
    for li in range(5):
        lp = params["layers"][li]
        xx = jnp.concatenate([X, nfe], axis=1)
        x_i = xx[dst]
        x_j = xx[src]
        k = _apply(lp["key"], jnp.concatenate([x_i, edge_emb], axis=1)).reshape(-1, HEADS, DPH)
        m = _apply(lp["msg"], jnp.concatenate([x_j, edge_emb], axis=1)).reshape(-1, HEADS, DPH)
        q = _apply(lp["query"], x_i).reshape(-1, HEADS, DPH) / jnp.sqrt(float(DPH))
        scores = (q * k).sum(axis=2)
        ex = jnp.exp(scores)
        den = jax.ops.segment_sum(ex, src, num_segments=N)
        deg = jax.ops.segment_sum(jnp.ones(src.shape[0], jnp.float32), src, num_segments=N)
        alpha = ex / (den[src] + 1e-16) * deg[src][:, None]
        out_msg = (m * alpha[:, :, None]).reshape(-1, HID)
        aggr = jax.ops.segment_sum(out_msg, dst, num_segments=N)
        X = _post_attention(aggr, lp, params["Wq"], km, vm, noises[li])
    return X
